# baseline (speedup 1.0000x reference)
import functools

import jax
import jax.numpy as jnp
from jax.experimental import pallas as pl
from jax.experimental.pallas import tpu as pltpu

_HW = 256
_SPATIAL = 4096
_TAPS3 = tuple((dz, dy, dx) for dz in (-1, 0, 1) for dy in (-1, 0, 1)
               for dx in (-1, 0, 1))
_TAPS5 = tuple((dz, dy, dx) for dz in (-2, -1, 0) for dy in (-2, -1, 0)
               for dx in (-2, -1, 0))


def _caps_kernel(u_ref, w_ref, o_ref, uhat_ref, col_ref, *,
                 t0, z0, t1, z1, routing, taps, post, halo, S):
    u = u_ref[0]
    lane = jax.lax.broadcasted_iota(jnp.int32, (1, S), 1)
    xg = lane & 15
    yg = (lane >> 4) & 15

    masks = []
    for (dz, dy, dx) in taps:
        if dy == 0 and dx == 0:
            masks.append(None)
            continue
        xv = xg + dx
        yv = yg + dy
        m = (xv >= 0) & (xv < 16) & (yv >= 0) & (yv < 16)
        masks.append(m.astype(jnp.float32))

    big = z0 % 8 == 0
    v = None
    for g in range(t0):
        pieces = []
        for ti, (dz, dy, dx) in enumerate(taps):
            s = dz * _HW + dy * 16 + dx
            src = u[g * z0:(g + 1) * z0, halo + s:halo + s + S]
            if masks[ti] is not None:
                src = src * masks[ti]
            if big:
                col_ref[ti * z0:(ti + 1) * z0, :] = src
            else:
                pieces.append(src)
        if big:
            col = col_ref[...]
        else:
            col = jnp.concatenate(pieces, axis=0)
        mm = jnp.dot(w_ref[g], col.astype(jnp.bfloat16),
                     preferred_element_type=jnp.float32)
        if routing == 0:
            o_ref[...] = jnp.maximum(mm, 0.0)
        else:
            uhat_ref[g] = mm.reshape(t1, z1, S)

    if routing == 0:
        return
    b = [jnp.zeros((t1, S), jnp.float32) for _ in range(t0)]
    for d in range(routing):
        c = [jax.nn.softmax(bg, axis=0) for bg in b]
        sacc = None
        for g in range(t0):
            term = c[g][:, None, :] * uhat_ref[g]
            sacc = term if sacc is None else sacc + term
        n2 = jnp.sum(sacc * sacc, axis=1, keepdims=True)
        v = (n2 / (1.0 + n2)) * sacc * jax.lax.rsqrt(n2 + 1e-9)
        if d < routing - 1:
            b = [bg + jnp.sum(uhat_ref[g] * v, axis=1)
                 for g, bg in enumerate(b)]
    if post == 'norm':
        vv = v[0]
        o_ref[...] = jnp.sqrt(jnp.sum(vv * vv, axis=0, keepdims=True) + 1e-9)
    else:
        o_ref[...] = v.reshape(t1 * z1, S)


def _layer(u_flat, w_mat, t0, z0, t1, z1, routing, taps=_TAPS3,
           post='caps', n_slabs=4, halo=512):
    cin = t0 * z0
    S = _SPATIAL // n_slabs
    W = S + 2 * halo
    up = jnp.pad(u_flat, ((0, 0), (halo, halo)))
    u_ov = jnp.stack([up[:, i * S:i * S + W] for i in range(n_slabs)], axis=0)
    m_out = 1 if post == 'norm' else t1 * z1
    nt = len(taps)
    col_shape = (nt * z0, S) if z0 % 8 == 0 else (8, 128)
    body = functools.partial(_caps_kernel, t0=t0, z0=z0, t1=t1, z1=z1,
                             routing=routing, taps=taps, post=post,
                             halo=halo, S=S)
    return pl.pallas_call(
        body,
        grid=(n_slabs,),
        in_specs=[
            pl.BlockSpec((1, cin, W), lambda i: (i, 0, 0)),
            pl.BlockSpec((t0, t1 * z1, nt * z0), lambda i: (0, 0, 0)),
        ],
        out_specs=pl.BlockSpec((m_out, S), lambda i: (0, i)),
        out_shape=jax.ShapeDtypeStruct((m_out, _SPATIAL), jnp.float32),
        scratch_shapes=[
            pltpu.VMEM((t0, t1, z1, S), jnp.float32),
            pltpu.VMEM(col_shape, jnp.float32),
        ],
        compiler_params=pltpu.CompilerParams(
            dimension_semantics=("arbitrary",),
            vmem_limit_bytes=56 * 1024 * 1024,
        ),
    )(u_ov, w_mat)


def _prep_w(w, t0, t1z1, z0, nt=27, flip=False):
    if flip:
        w = jnp.flip(w, axis=(-3, -2, -1))
    wm = w.reshape(t0, t1z1, z0, nt)
    wm = wm.transpose(0, 1, 3, 2).reshape(t0, t1z1, nt * z0)
    return wm.astype(jnp.bfloat16)


def kernel(x, w_conv1, w1a, w1b, w2a, w2b, w3a, w3b, w4, w5, w6, w7, w8, w10):
    xe = jnp.pad(x[0], ((0, 0), (0, 2), (0, 2), (0, 2))).reshape(4, _SPATIAL)
    h = _layer(xe, _prep_w(w_conv1.reshape(1, 16, 4, 3, 3, 3), 1, 16, 4),
               t0=1, z0=4, t1=1, z1=16, routing=0, taps=_TAPS5, halo=1024,
               n_slabs=1)
    skip1 = h
    u = _layer(h, _prep_w(w1a, 1, 32, 16), 1, 16, 2, 16, 1, n_slabs=1)
    u = _layer(u, _prep_w(w1b, 2, 64, 16), 2, 16, 4, 16, 3, n_slabs=1)
    skip2 = u
    u = _layer(u, _prep_w(w2a, 4, 128, 16), 4, 16, 4, 32, 3, n_slabs=1)
    u = _layer(u, _prep_w(w2b, 4, 256, 32), 4, 32, 8, 32, 3, n_slabs=1)
    skip3 = u
    u = _layer(u, _prep_w(w3a, 8, 512, 32), 8, 32, 8, 64, 3, n_slabs=4)
    u = _layer(u, _prep_w(w3b, 8, 256, 64), 8, 64, 8, 32, 3, n_slabs=4)
    u = _layer(u, _prep_w(w4, 8, 256, 32, flip=True), 8, 32, 8, 32, 3,
               n_slabs=2)
    u = jnp.concatenate([u, skip3], axis=0)
    u = _layer(u, _prep_w(w5, 16, 128, 32), 16, 32, 4, 32, 3, n_slabs=2)
    u = _layer(u, _prep_w(w6, 4, 64, 32, flip=True), 4, 32, 4, 16, 3,
               n_slabs=1)
    u = jnp.concatenate([u, skip2], axis=0)
    u = _layer(u, _prep_w(w7, 8, 64, 16), 8, 16, 4, 16, 3, n_slabs=1)
    u = _layer(u, _prep_w(w8, 4, 32, 16, flip=True), 4, 16, 2, 16, 3,
               n_slabs=1)
    u = jnp.concatenate([u, skip1], axis=0)
    out = _layer(u, _prep_w(w10, 3, 16, 16), 3, 16, 1, 16, 3, post='norm',
                 n_slabs=1)
    return out.reshape(1, 1, 16, 16, 16)


# speedup vs baseline: 1.1170x; 1.1170x over previous
import functools

import jax
import jax.numpy as jnp
from jax.experimental import pallas as pl
from jax.experimental.pallas import tpu as pltpu

_HW = 256
_SPATIAL = 4096
_TAPS3 = tuple((dz, dy, dx) for dz in (-1, 0, 1) for dy in (-1, 0, 1)
               for dx in (-1, 0, 1))
_TAPS5 = tuple((dz, dy, dx) for dz in (-2, -1, 0) for dy in (-2, -1, 0)
               for dx in (-2, -1, 0))


def _caps_kernel(u_ref, w_ref, o_ref, uhat_ref, col_ref, *,
                 t0, z0, t1, z1, routing, taps, post, halo, S):
    u = u_ref[0]
    lane = jax.lax.broadcasted_iota(jnp.int32, (1, S), 1)
    xg = lane & 15
    yg = (lane >> 4) & 15

    masks = []
    for (dz, dy, dx) in taps:
        if dy == 0 and dx == 0:
            masks.append(None)
            continue
        xv = xg + dx
        yv = yg + dy
        m = (xv >= 0) & (xv < 16) & (yv >= 0) & (yv < 16)
        masks.append(m.astype(jnp.float32))

    big = z0 % 8 == 0
    v = None
    for g in range(t0):
        pieces = []
        for ti, (dz, dy, dx) in enumerate(taps):
            s = dz * _HW + dy * 16 + dx
            src = u[g * z0:(g + 1) * z0, halo + s:halo + s + S]
            if masks[ti] is not None:
                src = src * masks[ti]
            if big:
                col_ref[ti * z0:(ti + 1) * z0, :] = src
            else:
                pieces.append(src)
        if big:
            col = col_ref[...]
        else:
            col = jnp.concatenate(pieces, axis=0)
        mm = jnp.dot(w_ref[g], col.astype(jnp.bfloat16),
                     preferred_element_type=jnp.float32)
        if routing == 0:
            o_ref[...] = jnp.maximum(mm, 0.0)
        else:
            uhat_ref[g] = mm.reshape(t1, z1, S)

    if routing == 0:
        return
    b = [jnp.zeros((t1, S), jnp.float32) for _ in range(t0)]
    for d in range(routing):
        c = [jax.nn.softmax(bg, axis=0) for bg in b]
        sacc = None
        for g in range(t0):
            term = c[g][:, None, :] * uhat_ref[g]
            sacc = term if sacc is None else sacc + term
        n2 = jnp.sum(sacc * sacc, axis=1, keepdims=True)
        v = (n2 / (1.0 + n2)) * sacc * jax.lax.rsqrt(n2 + 1e-9)
        if d < routing - 1:
            b = [bg + jnp.sum(uhat_ref[g] * v, axis=1)
                 for g, bg in enumerate(b)]
    if post == 'norm':
        vv = v[0]
        o_ref[...] = jnp.sqrt(jnp.sum(vv * vv, axis=0, keepdims=True) + 1e-9)
    else:
        o_ref[...] = v.reshape(t1 * z1, S)


def _layer(u_flat, w_mat, t0, z0, t1, z1, routing, taps=_TAPS3,
           post='caps', n_slabs=4, halo=512):
    cin = t0 * z0
    S = _SPATIAL // n_slabs
    W = S + 2 * halo
    up = jnp.pad(u_flat, ((0, 0), (halo, halo)))
    u_ov = jnp.stack([up[:, i * S:i * S + W] for i in range(n_slabs)], axis=0)
    m_out = 1 if post == 'norm' else t1 * z1
    nt = len(taps)
    col_shape = (nt * z0, S) if z0 % 8 == 0 else (8, 128)
    body = functools.partial(_caps_kernel, t0=t0, z0=z0, t1=t1, z1=z1,
                             routing=routing, taps=taps, post=post,
                             halo=halo, S=S)
    return pl.pallas_call(
        body,
        grid=(n_slabs,),
        in_specs=[
            pl.BlockSpec((1, cin, W), lambda i: (i, 0, 0)),
            pl.BlockSpec((t0, t1 * z1, nt * z0), lambda i: (0, 0, 0)),
        ],
        out_specs=pl.BlockSpec((m_out, S), lambda i: (0, i)),
        out_shape=jax.ShapeDtypeStruct((m_out, _SPATIAL), jnp.float32),
        scratch_shapes=[
            pltpu.VMEM((t0, t1, z1, S), jnp.float32),
            pltpu.VMEM(col_shape, jnp.float32),
        ],
        compiler_params=pltpu.CompilerParams(
            dimension_semantics=("arbitrary",),
            vmem_limit_bytes=56 * 1024 * 1024,
        ),
    )(u_ov, w_mat)


def _prep_w(w, t0, t1z1, z0, nt=27, flip=False):
    if flip:
        w = jnp.flip(w, axis=(-3, -2, -1))
    wm = w.reshape(t0, t1z1, z0, nt)
    wm = wm.transpose(0, 1, 3, 2).reshape(t0, t1z1, nt * z0)
    return wm.astype(jnp.bfloat16)


def kernel(x, w_conv1, w1a, w1b, w2a, w2b, w3a, w3b, w4, w5, w6, w7, w8, w10):
    xe = jnp.pad(x[0], ((0, 0), (0, 2), (0, 2), (0, 2))).reshape(4, _SPATIAL)
    h = _layer(xe, _prep_w(w_conv1.reshape(1, 16, 4, 3, 3, 3), 1, 16, 4),
               t0=1, z0=4, t1=1, z1=16, routing=0, taps=_TAPS5, halo=1024, n_slabs=4)
    skip1 = h
    u = _layer(h, _prep_w(w1a, 1, 32, 16), 1, 16, 2, 16, 1, n_slabs=4)
    u = _layer(u, _prep_w(w1b, 2, 64, 16), 2, 16, 4, 16, 3, n_slabs=4)
    skip2 = u
    u = _layer(u, _prep_w(w2a, 4, 128, 16), 4, 16, 4, 32, 3, n_slabs=4)
    u = _layer(u, _prep_w(w2b, 4, 256, 32), 4, 32, 8, 32, 3, n_slabs=4)
    skip3 = u
    u = _layer(u, _prep_w(w3a, 8, 512, 32), 8, 32, 8, 64, 3, n_slabs=8)
    u = _layer(u, _prep_w(w3b, 8, 256, 64), 8, 64, 8, 32, 3, n_slabs=8)
    u = _layer(u, _prep_w(w4, 8, 256, 32, flip=True), 8, 32, 8, 32, 3,
               n_slabs=8)
    u = jnp.concatenate([u, skip3], axis=0)
    u = _layer(u, _prep_w(w5, 16, 128, 32), 16, 32, 4, 32, 3, n_slabs=8)
    u = _layer(u, _prep_w(w6, 4, 64, 32, flip=True), 4, 32, 4, 16, 3,
               n_slabs=4)
    u = jnp.concatenate([u, skip2], axis=0)
    u = _layer(u, _prep_w(w7, 8, 64, 16), 8, 16, 4, 16, 3, n_slabs=4)
    u = _layer(u, _prep_w(w8, 4, 32, 16, flip=True), 4, 16, 2, 16, 3,
               n_slabs=4)
    u = jnp.concatenate([u, skip1], axis=0)
    out = _layer(u, _prep_w(w10, 3, 16, 16), 3, 16, 1, 16, 3, post='norm',
                 n_slabs=4)
    return out.reshape(1, 1, 16, 16, 16)
